# Initial kernel scaffold; baseline (speedup 1.0000x reference)
import jax
import jax.numpy as jnp
from jax.experimental import pallas as pl


def kernel(image, clusters, W1, b1, W2, b2):
    raise NotImplementedError("write your pallas kernel here")


# speedup vs baseline: 1.1875x; 1.1875x over previous
import functools

import jax
import jax.numpy as jnp
from jax.experimental import pallas as pl


def _mlp_kernel(x_ref, cl_ref, w1t_ref, b1_ref, w2t_ref, b2t_ref, out_ref,
                *, classes, features, hs):
    x = x_ref[0]
    cl = cl_ref[0, 0]
    t = x.shape[-1]
    cf = classes * features

    h = jax.lax.dot_general(
        w1t_ref[...], x,
        dimension_numbers=(((1,), (0,)), ((), ())),
        preferred_element_type=jnp.float32,
    ) + b1_ref[...]
    h = jnp.maximum(h, 0.0)

    row_class = jax.lax.broadcasted_iota(jnp.int32, (cf, t), 0) // features
    h = jnp.where(row_class == cl[None, :], h, 0.0)

    y = jax.lax.dot_general(
        w2t_ref[...], h,
        dimension_numbers=(((1,), (0,)), ((), ())),
        preferred_element_type=jnp.float32,
    )

    onehot = (jax.lax.broadcasted_iota(jnp.int32, (classes, t), 0)
              == cl[None, :]).astype(jnp.float32)
    y = y + jax.lax.dot_general(
        b2t_ref[...], onehot,
        dimension_numbers=(((1,), (0,)), ((), ())),
        preferred_element_type=jnp.float32,
    )
    out_ref[0] = jnp.maximum(y, 0.0)


def kernel(image, clusters, W1, b1, W2, b2):
    B, C, H, W = image.shape
    classes, _, features = W1.shape
    hs = W2.shape[-1]
    N = H * W
    T = 2048

    x = image.reshape(B, C, N)
    cl = clusters.reshape(B, 1, N).astype(jnp.int32)

    w1t = jnp.transpose(W1, (1, 0, 2)).reshape(C, classes * features).T
    b1c = b1.reshape(classes * features, 1)
    w2t = W2.reshape(classes * features, hs).T
    b2t = b2.T

    grid = (B, N // T)
    out = pl.pallas_call(
        functools.partial(_mlp_kernel, classes=classes, features=features, hs=hs),
        grid=grid,
        in_specs=[
            pl.BlockSpec((1, C, T), lambda b, t: (b, 0, t)),
            pl.BlockSpec((1, 1, T), lambda b, t: (b, 0, t)),
            pl.BlockSpec((classes * features, C), lambda b, t: (0, 0)),
            pl.BlockSpec((classes * features, 1), lambda b, t: (0, 0)),
            pl.BlockSpec((hs, classes * features), lambda b, t: (0, 0)),
            pl.BlockSpec((hs, classes), lambda b, t: (0, 0)),
        ],
        out_specs=pl.BlockSpec((1, hs, T), lambda b, t: (b, 0, t)),
        out_shape=jax.ShapeDtypeStruct((B, hs, N), image.dtype),
    )(x, cl, w1t, b1c, w2t, b2t)

    return out.reshape(B, hs, H, W)


# speedup vs baseline: 1.2798x; 1.0777x over previous
import functools

import jax
import jax.numpy as jnp
from jax.experimental import pallas as pl


def _mlp_kernel(x_ref, cl_ref, s1_ref, w2_ref, out_ref, *, classes, c_in, hs):
    x = x_ref[0].astype(jnp.bfloat16)
    cl = cl_ref[0, 0]
    t = x.shape[-1]

    xt = jnp.concatenate([x] * classes, axis=0)
    row_class = jax.lax.broadcasted_iota(jnp.int32, (classes * c_in, t), 0) // c_in
    xm = jnp.where(row_class == cl[None, :], xt, jnp.bfloat16(0.0))
    onehot = (jax.lax.broadcasted_iota(jnp.int32, (classes, t), 0)
              == cl[None, :]).astype(jnp.bfloat16)
    xm = jnp.concatenate([xm, onehot], axis=0)

    h = jax.lax.dot_general(
        s1_ref[...], xm,
        dimension_numbers=(((1,), (0,)), ((), ())),
        preferred_element_type=jnp.float32,
    )
    h = jnp.maximum(h, 0.0).astype(jnp.bfloat16)
    hm = jnp.concatenate([h, onehot], axis=0)

    y_all = jax.lax.dot_general(
        w2_ref[...], hm,
        dimension_numbers=(((1,), (0,)), ((), ())),
        preferred_element_type=jnp.float32,
    )

    y = jnp.zeros((32, t), dtype=jnp.float32)
    for l in range(classes):
        m = (cl[None, :] == l)
        y = y + jnp.where(m, y_all[32 * l:32 * (l + 1), :], 0.0)
    out_ref[0] = jnp.maximum(y[:hs, :], 0.0)


def kernel(image, clusters, W1, b1, W2, b2):
    B, C, H, W = image.shape
    classes, _, features = W1.shape
    hs = W2.shape[-1]
    N = H * W
    T = 2048

    x = image.reshape(B, C, N)
    cl = clusters.reshape(B, 1, N).astype(jnp.int32)

    s1 = jnp.concatenate([W1.reshape(classes * C, features), b1], axis=0)
    s1 = s1.T.astype(jnp.bfloat16)

    w2wide = jnp.zeros((features + classes, classes * 32), dtype=jnp.float32)
    for l in range(classes):
        w2wide = w2wide.at[:features, 32 * l:32 * l + hs].set(W2[l])
        w2wide = w2wide.at[features + l, 32 * l:32 * l + hs].set(b2[l])
    w2a = w2wide.T.astype(jnp.bfloat16)

    grid = (B, N // T)
    out = pl.pallas_call(
        functools.partial(_mlp_kernel, classes=classes, c_in=C, hs=hs),
        grid=grid,
        in_specs=[
            pl.BlockSpec((1, C, T), lambda b, t: (b, 0, t)),
            pl.BlockSpec((1, 1, T), lambda b, t: (b, 0, t)),
            pl.BlockSpec(s1.shape, lambda b, t: (0, 0)),
            pl.BlockSpec(w2a.shape, lambda b, t: (0, 0)),
        ],
        out_specs=pl.BlockSpec((1, hs, T), lambda b, t: (b, 0, t)),
        out_shape=jax.ShapeDtypeStruct((B, hs, N), image.dtype),
    )(x, cl, s1, w2a)

    return out.reshape(B, hs, H, W)


# speedup vs baseline: 1.6967x; 1.3258x over previous
import functools

import jax
import jax.numpy as jnp
from jax.experimental import pallas as pl


def _mlp_kernel(x_ref, cl_ref, s1_ref, w2_ref, out_ref, *, classes, c_in, hs):
    x = x_ref[0].astype(jnp.bfloat16)
    cl = cl_ref[0, 0]
    t = x.shape[-1]

    xt = jnp.concatenate([x] * classes, axis=0)
    row_class = jax.lax.broadcasted_iota(jnp.int32, (classes * c_in, t), 0) // c_in
    xm = jnp.where(row_class == cl[None, :], xt, jnp.bfloat16(0.0))
    onehot = (jax.lax.broadcasted_iota(jnp.int32, (classes, t), 0)
              == cl[None, :]).astype(jnp.bfloat16)
    xm = jnp.concatenate([xm, onehot], axis=0)

    h = jax.lax.dot_general(
        s1_ref[...], xm,
        dimension_numbers=(((1,), (0,)), ((), ())),
        preferred_element_type=jnp.float32,
    )
    h = jnp.maximum(h, 0.0).astype(jnp.bfloat16)
    hm = jnp.concatenate([h, onehot], axis=0)

    y_all = jax.lax.dot_general(
        w2_ref[...], hm,
        dimension_numbers=(((1,), (0,)), ((), ())),
        preferred_element_type=jnp.float32,
    )

    y = jnp.zeros((32, t), dtype=jnp.float32)
    for l in range(classes):
        m = (cl[None, :] == l)
        y = y + jnp.where(m, y_all[32 * l:32 * (l + 1), :], 0.0)
    out_ref[0] = jnp.maximum(y[:hs, :], 0.0)


def kernel(image, clusters, W1, b1, W2, b2):
    B, C, H, W = image.shape
    classes, _, features = W1.shape
    hs = W2.shape[-1]
    N = H * W
    T = 4096

    x = image.reshape(B, C, N)
    cl = clusters.reshape(B, 1, N).astype(jnp.int32)

    s1 = jnp.concatenate([W1.reshape(classes * C, features), b1], axis=0)
    s1 = s1.T.astype(jnp.bfloat16)

    w2wide = jnp.zeros((features + classes, classes * 32), dtype=jnp.float32)
    for l in range(classes):
        w2wide = w2wide.at[:features, 32 * l:32 * l + hs].set(W2[l])
        w2wide = w2wide.at[features + l, 32 * l:32 * l + hs].set(b2[l])
    w2a = w2wide.T.astype(jnp.bfloat16)

    grid = (B, N // T)
    out = pl.pallas_call(
        functools.partial(_mlp_kernel, classes=classes, c_in=C, hs=hs),
        grid=grid,
        in_specs=[
            pl.BlockSpec((1, C, T), lambda b, t: (b, 0, t)),
            pl.BlockSpec((1, 1, T), lambda b, t: (b, 0, t)),
            pl.BlockSpec(s1.shape, lambda b, t: (0, 0)),
            pl.BlockSpec(w2a.shape, lambda b, t: (0, 0)),
        ],
        out_specs=pl.BlockSpec((1, hs, T), lambda b, t: (b, 0, t)),
        out_shape=jax.ShapeDtypeStruct((B, hs, N), image.dtype),
    )(x, cl, s1, w2a)

    return out.reshape(B, hs, H, W)


# speedup vs baseline: 3.0351x; 1.7888x over previous
import functools

import jax
import jax.numpy as jnp
from jax.experimental import pallas as pl


def _mlp_kernel(x_ref, cl_ref, s1_ref, w2_ref, out_ref, *, classes, c_in, hs):
    hb, w = x_ref.shape[2], x_ref.shape[3]
    t = hb * w
    x = x_ref[0].reshape(c_in, t).astype(jnp.bfloat16)
    cl = cl_ref[0, 0].reshape(1, t)

    xt = jnp.concatenate([x] * classes, axis=0)
    row_class = jax.lax.broadcasted_iota(jnp.int32, (classes * c_in, t), 0) // c_in
    xm = jnp.where(row_class == cl, xt, jnp.bfloat16(0.0))
    onehot = (jax.lax.broadcasted_iota(jnp.int32, (classes, t), 0)
              == cl).astype(jnp.bfloat16)
    xm = jnp.concatenate([xm, onehot], axis=0)

    h = jax.lax.dot_general(
        s1_ref[...], xm,
        dimension_numbers=(((1,), (0,)), ((), ())),
        preferred_element_type=jnp.float32,
    )
    h = jnp.maximum(h, 0.0).astype(jnp.bfloat16)
    hm = jnp.concatenate([h, onehot], axis=0)

    y_all = jax.lax.dot_general(
        w2_ref[...], hm,
        dimension_numbers=(((1,), (0,)), ((), ())),
        preferred_element_type=jnp.float32,
    )

    y = jnp.zeros((32, t), dtype=jnp.float32)
    for l in range(classes):
        y = y + jnp.where(cl == l, y_all[32 * l:32 * (l + 1), :], 0.0)
    y = jnp.maximum(y[:hs, :], 0.0)
    out_ref[0] = y.reshape(hs, hb, w)


def kernel(image, clusters, W1, b1, W2, b2):
    B, C, H, W = image.shape
    classes, _, features = W1.shape
    hs = W2.shape[-1]
    HB = 8

    cl = clusters.astype(jnp.int32)

    s1 = jnp.concatenate([W1.reshape(classes * C, features), b1], axis=0)
    s1 = s1.T.astype(jnp.bfloat16)

    w2wide = jnp.zeros((features + classes, classes * 32), dtype=jnp.float32)
    for l in range(classes):
        w2wide = w2wide.at[:features, 32 * l:32 * l + hs].set(W2[l])
        w2wide = w2wide.at[features + l, 32 * l:32 * l + hs].set(b2[l])
    w2a = w2wide.T.astype(jnp.bfloat16)

    grid = (B, H // HB)
    out = pl.pallas_call(
        functools.partial(_mlp_kernel, classes=classes, c_in=C, hs=hs),
        grid=grid,
        in_specs=[
            pl.BlockSpec((1, C, HB, W), lambda b, t: (b, 0, t, 0)),
            pl.BlockSpec((1, 1, HB, W), lambda b, t: (b, 0, t, 0)),
            pl.BlockSpec(s1.shape, lambda b, t: (0, 0)),
            pl.BlockSpec(w2a.shape, lambda b, t: (0, 0)),
        ],
        out_specs=pl.BlockSpec((1, hs, HB, W), lambda b, t: (b, 0, t, 0)),
        out_shape=jax.ShapeDtypeStruct((B, hs, H, W), image.dtype),
    )(image, cl, s1, w2a)

    return out


# speedup vs baseline: 3.8347x; 1.2634x over previous
import functools

import jax
import jax.numpy as jnp
from jax.experimental import pallas as pl


def _mlp_kernel(x_ref, cl_ref, s1_ref, w2_ref, out_ref, *, classes, c_in, hs):
    hb, w = x_ref.shape[2], x_ref.shape[3]
    t = hb * w
    x = x_ref[0].astype(jnp.bfloat16).reshape(c_in, t)
    cl = cl_ref[0, 0].reshape(1, t)

    xt = jnp.concatenate([x] * classes, axis=0)
    row_class = jax.lax.broadcasted_iota(jnp.int32, (classes * c_in, t), 0) // c_in
    xm = jnp.where(row_class == cl, xt, jnp.bfloat16(0.0))
    onehot = (jax.lax.broadcasted_iota(jnp.int32, (classes, t), 0)
              == cl).astype(jnp.bfloat16)
    xm = jnp.concatenate([xm, onehot], axis=0)

    h = jax.lax.dot_general(
        s1_ref[...], xm,
        dimension_numbers=(((1,), (0,)), ((), ())),
        preferred_element_type=jnp.float32,
    )
    h = jnp.maximum(h, 0.0).astype(jnp.bfloat16)
    hm = jnp.concatenate([h, onehot], axis=0)

    y_all = jax.lax.dot_general(
        w2_ref[...], hm,
        dimension_numbers=(((1,), (0,)), ((), ())),
        preferred_element_type=jnp.float32,
    ).astype(jnp.bfloat16)

    y = jnp.zeros((32, t), dtype=jnp.bfloat16)
    for l in range(classes):
        y = y + jnp.where(cl == l, y_all[32 * l:32 * (l + 1), :], jnp.bfloat16(0.0))
    y = jnp.maximum(y[:hs, :], jnp.bfloat16(0.0))
    out_ref[0] = y.reshape(hs, hb, w).astype(jnp.float32)


def kernel(image, clusters, W1, b1, W2, b2):
    B, C, H, W = image.shape
    classes, _, features = W1.shape
    hs = W2.shape[-1]
    HB = 16

    cl = clusters.astype(jnp.int32)

    s1 = jnp.concatenate([W1.reshape(classes * C, features), b1], axis=0)
    s1 = s1.T.astype(jnp.bfloat16)

    w2wide = jnp.zeros((features + classes, classes * 32), dtype=jnp.float32)
    for l in range(classes):
        w2wide = w2wide.at[:features, 32 * l:32 * l + hs].set(W2[l])
        w2wide = w2wide.at[features + l, 32 * l:32 * l + hs].set(b2[l])
    w2a = w2wide.T.astype(jnp.bfloat16)

    grid = (B, H // HB)
    out = pl.pallas_call(
        functools.partial(_mlp_kernel, classes=classes, c_in=C, hs=hs),
        grid=grid,
        in_specs=[
            pl.BlockSpec((1, C, HB, W), lambda b, t: (b, 0, t, 0)),
            pl.BlockSpec((1, 1, HB, W), lambda b, t: (b, 0, t, 0)),
            pl.BlockSpec(s1.shape, lambda b, t: (0, 0)),
            pl.BlockSpec(w2a.shape, lambda b, t: (0, 0)),
        ],
        out_specs=pl.BlockSpec((1, hs, HB, W), lambda b, t: (b, 0, t, 0)),
        out_shape=jax.ShapeDtypeStruct((B, hs, H, W), image.dtype),
    )(image, cl, s1, w2a)

    return out


# speedup vs baseline: 4.0317x; 1.0514x over previous
import functools

import jax
import jax.numpy as jnp
from jax.experimental import pallas as pl


def _mlp_kernel(x_ref, cl_ref, s1_ref, w2_ref, out_ref, *, classes, c_in, hs):
    hb, w = x_ref.shape[2], x_ref.shape[3]
    t = hb * w
    x = x_ref[0].astype(jnp.bfloat16).reshape(c_in, t)
    cl = cl_ref[0, 0].reshape(1, t)

    xt = jnp.concatenate([x] * classes, axis=0)
    row_class = jax.lax.broadcasted_iota(jnp.int32, (classes * c_in, t), 0) // c_in
    xm = jnp.where(row_class == cl, xt, jnp.bfloat16(0.0))
    onehot = (jax.lax.broadcasted_iota(jnp.int32, (classes, t), 0)
              == cl).astype(jnp.bfloat16)
    xm = jnp.concatenate([xm, onehot], axis=0)

    h = jax.lax.dot_general(
        s1_ref[...], xm,
        dimension_numbers=(((1,), (0,)), ((), ())),
        preferred_element_type=jnp.float32,
    )
    h = jnp.maximum(h, 0.0).astype(jnp.bfloat16)
    hm = jnp.concatenate([h, onehot], axis=0)

    y_all = jax.lax.dot_general(
        w2_ref[...], hm,
        dimension_numbers=(((1,), (0,)), ((), ())),
        preferred_element_type=jnp.float32,
    ).astype(jnp.bfloat16)

    y = jnp.zeros((32, t), dtype=jnp.bfloat16)
    for l in range(classes):
        y = y + jnp.where(cl == l, y_all[32 * l:32 * (l + 1), :], jnp.bfloat16(0.0))
    y = jnp.maximum(y[:hs, :], jnp.bfloat16(0.0))
    out_ref[0] = y.reshape(hs, hb, w).astype(jnp.float32)


def kernel(image, clusters, W1, b1, W2, b2):
    B, C, H, W = image.shape
    classes, _, features = W1.shape
    hs = W2.shape[-1]
    HB = 32

    cl = clusters.astype(jnp.int32)

    s1 = jnp.concatenate([W1.reshape(classes * C, features), b1], axis=0)
    s1 = s1.T.astype(jnp.bfloat16)

    w2wide = jnp.zeros((features + classes, classes * 32), dtype=jnp.float32)
    for l in range(classes):
        w2wide = w2wide.at[:features, 32 * l:32 * l + hs].set(W2[l])
        w2wide = w2wide.at[features + l, 32 * l:32 * l + hs].set(b2[l])
    w2a = w2wide.T.astype(jnp.bfloat16)

    grid = (B, H // HB)
    out = pl.pallas_call(
        functools.partial(_mlp_kernel, classes=classes, c_in=C, hs=hs),
        grid=grid,
        in_specs=[
            pl.BlockSpec((1, C, HB, W), lambda b, t: (b, 0, t, 0)),
            pl.BlockSpec((1, 1, HB, W), lambda b, t: (b, 0, t, 0)),
            pl.BlockSpec(s1.shape, lambda b, t: (0, 0)),
            pl.BlockSpec(w2a.shape, lambda b, t: (0, 0)),
        ],
        out_specs=pl.BlockSpec((1, hs, HB, W), lambda b, t: (b, 0, t, 0)),
        out_shape=jax.ShapeDtypeStruct((B, hs, H, W), image.dtype),
    )(image, cl, s1, w2a)

    return out


# speedup vs baseline: 4.1101x; 1.0194x over previous
import functools

import jax
import jax.numpy as jnp
from jax.experimental import pallas as pl


def _mlp_kernel(x_ref, cl_ref, s1_ref, w2_ref, out_ref, *, classes, c_in, hs):
    hb, w = x_ref.shape[2], x_ref.shape[3]
    t = hb * w
    x = x_ref[0].astype(jnp.bfloat16).reshape(c_in, t)
    cl = cl_ref[0, 0].reshape(1, t)

    xt = jnp.concatenate([x] * classes, axis=0)
    row_class = jax.lax.broadcasted_iota(jnp.int32, (classes * c_in, t), 0) // c_in
    xm = jnp.where(row_class == cl, xt, jnp.bfloat16(0.0))
    onehot = (jax.lax.broadcasted_iota(jnp.int32, (classes, t), 0)
              == cl).astype(jnp.bfloat16)
    xm = jnp.concatenate([xm, onehot], axis=0)

    h = jax.lax.dot_general(
        s1_ref[...], xm,
        dimension_numbers=(((1,), (0,)), ((), ())),
        preferred_element_type=jnp.float32,
    )
    h = jnp.maximum(h, 0.0).astype(jnp.bfloat16)
    hm = jnp.concatenate([h, onehot], axis=0)

    y_all = jax.lax.dot_general(
        w2_ref[...], hm,
        dimension_numbers=(((1,), (0,)), ((), ())),
        preferred_element_type=jnp.float32,
    ).astype(jnp.bfloat16)

    y = jnp.zeros((32, t), dtype=jnp.bfloat16)
    for l in range(classes):
        y = y + jnp.where(cl == l, y_all[32 * l:32 * (l + 1), :], jnp.bfloat16(0.0))
    y = jnp.maximum(y[:hs, :], jnp.bfloat16(0.0))
    out_ref[0] = y.reshape(hs, hb, w).astype(jnp.float32)


def kernel(image, clusters, W1, b1, W2, b2):
    B, C, H, W = image.shape
    classes, _, features = W1.shape
    hs = W2.shape[-1]
    HB = 64

    cl = clusters.astype(jnp.int32)

    s1 = jnp.concatenate([W1.reshape(classes * C, features), b1], axis=0)
    s1 = s1.T.astype(jnp.bfloat16)

    w2wide = jnp.zeros((features + classes, classes * 32), dtype=jnp.float32)
    for l in range(classes):
        w2wide = w2wide.at[:features, 32 * l:32 * l + hs].set(W2[l])
        w2wide = w2wide.at[features + l, 32 * l:32 * l + hs].set(b2[l])
    w2a = w2wide.T.astype(jnp.bfloat16)

    grid = (B, H // HB)
    out = pl.pallas_call(
        functools.partial(_mlp_kernel, classes=classes, c_in=C, hs=hs),
        grid=grid,
        in_specs=[
            pl.BlockSpec((1, C, HB, W), lambda b, t: (b, 0, t, 0)),
            pl.BlockSpec((1, 1, HB, W), lambda b, t: (b, 0, t, 0)),
            pl.BlockSpec(s1.shape, lambda b, t: (0, 0)),
            pl.BlockSpec(w2a.shape, lambda b, t: (0, 0)),
        ],
        out_specs=pl.BlockSpec((1, hs, HB, W), lambda b, t: (b, 0, t, 0)),
        out_shape=jax.ShapeDtypeStruct((B, hs, H, W), image.dtype),
    )(image, cl, s1, w2a)

    return out


# speedup vs baseline: 4.1511x; 1.0100x over previous
import functools

import jax
import jax.numpy as jnp
from jax.experimental import pallas as pl


def _mlp_kernel(x_ref, cl_ref, s1_ref, w2_ref, out_ref, *, classes, c_in, hs):
    hb, w = x_ref.shape[2], x_ref.shape[3]
    t = hb * w
    x = x_ref[0].astype(jnp.bfloat16).reshape(c_in, t)
    cl = cl_ref[0, 0].reshape(1, t)

    xt = jnp.concatenate([x] * classes, axis=0)
    row_class = jax.lax.broadcasted_iota(jnp.int32, (classes * c_in, t), 0) // c_in
    xm = jnp.where(row_class == cl, xt, jnp.bfloat16(0.0))
    onehot = (jax.lax.broadcasted_iota(jnp.int32, (classes, t), 0)
              == cl).astype(jnp.bfloat16)
    xm = jnp.concatenate([xm, onehot], axis=0)

    h = jax.lax.dot_general(
        s1_ref[...], xm,
        dimension_numbers=(((1,), (0,)), ((), ())),
        preferred_element_type=jnp.float32,
    )
    h = jnp.maximum(h, 0.0).astype(jnp.bfloat16)
    hm = jnp.concatenate([h, onehot], axis=0)

    y_all = jax.lax.dot_general(
        w2_ref[...], hm,
        dimension_numbers=(((1,), (0,)), ((), ())),
        preferred_element_type=jnp.float32,
    ).astype(jnp.bfloat16)

    y = jnp.zeros((32, t), dtype=jnp.bfloat16)
    for l in range(classes):
        y = y + y_all[32 * l:32 * (l + 1), :] * onehot[l:l + 1, :]
    y = jnp.maximum(y[:hs, :], jnp.bfloat16(0.0))
    out_ref[0] = y.reshape(hs, hb, w).astype(jnp.float32)


def kernel(image, clusters, W1, b1, W2, b2):
    B, C, H, W = image.shape
    classes, _, features = W1.shape
    hs = W2.shape[-1]
    HB = 64

    cl = clusters.astype(jnp.int32)

    s1 = jnp.concatenate([W1.reshape(classes * C, features), b1], axis=0)
    s1 = s1.T.astype(jnp.bfloat16)

    w2wide = jnp.zeros((features + classes, classes * 32), dtype=jnp.float32)
    for l in range(classes):
        w2wide = w2wide.at[:features, 32 * l:32 * l + hs].set(W2[l])
        w2wide = w2wide.at[features + l, 32 * l:32 * l + hs].set(b2[l])
    w2a = w2wide.T.astype(jnp.bfloat16)

    grid = (B, H // HB)
    out = pl.pallas_call(
        functools.partial(_mlp_kernel, classes=classes, c_in=C, hs=hs),
        grid=grid,
        in_specs=[
            pl.BlockSpec((1, C, HB, W), lambda b, t: (b, 0, t, 0)),
            pl.BlockSpec((1, 1, HB, W), lambda b, t: (b, 0, t, 0)),
            pl.BlockSpec(s1.shape, lambda b, t: (0, 0)),
            pl.BlockSpec(w2a.shape, lambda b, t: (0, 0)),
        ],
        out_specs=pl.BlockSpec((1, hs, HB, W), lambda b, t: (b, 0, t, 0)),
        out_shape=jax.ShapeDtypeStruct((B, hs, H, W), image.dtype),
    )(image, cl, s1, w2a)

    return out
